# baseline (device time: 130850 ns/iter reference)
import jax
import jax.numpy as jnp
from jax import lax
from jax.experimental import pallas as pl
from jax.experimental.pallas import tpu as pltpu

N_DEV = 16
T = 1024
D = 512
H = 1024
E = 64
E_LOC = 4
CAP = 204
CAP_PAD = 208

C_SEND, C_RECV, X_SEND, X_RECV, M_SEND, M_RECV, Y_SEND, Y_RECV = range(8)
LOGICAL = pl.DeviceIdType.LOGICAL


def kernel(x, router_W, route_idx, expert_W):
    del router_W

    def body(x_ref, route_ref, w_ref, out_ref,
             counts_mine, counts_vmem, stage_vmem, meta_src, xbuf, ybuf,
             meta_vmem, route_smem, stage_smem, meta_smem, rank_ctr,
             sems, local_sem, exit_sem):
        my = lax.axis_index("i")

        out_ref[...] = jnp.zeros((T, H), jnp.float32)
        rt = route_ref[...]
        eids = lax.broadcasted_iota(jnp.int32, (T, E), 1)
        counts_mine[...] = jnp.sum((rt == eids).astype(jnp.int32), axis=0,
                                   keepdims=True)
        meta_src[...] = my * T + lax.broadcasted_iota(jnp.int32, (T, 1), 0)

        rcopy = pltpu.make_async_copy(route_ref, route_smem, local_sem)
        rcopy.start()

        def zero_ctr(e_, c):
            rank_ctr[e_] = 0
            return c
        lax.fori_loop(0, E, zero_ctr, 0)

        barrier_sem = pltpu.get_barrier_semaphore()

        def bsig(d, c):
            @pl.when(d != my)
            def _():
                pl.semaphore_signal(barrier_sem, 1, device_id=d,
                                    device_id_type=LOGICAL)
            return c
        lax.fori_loop(0, N_DEV, bsig, 0)
        pl.semaphore_wait(barrier_sem, N_DEV - 1)

        def csend(d, c):
            pltpu.make_async_remote_copy(
                counts_mine, counts_vmem.at[pl.ds(my, 1), :],
                sems.at[C_SEND], sems.at[C_RECV],
                device_id=d, device_id_type=LOGICAL,
            ).start()
            return c
        lax.fori_loop(0, N_DEV, csend, 0)

        c_wait = pltpu.make_async_remote_copy(
            counts_mine, counts_vmem.at[pl.ds(0, 1), :],
            sems.at[C_SEND], sems.at[C_RECV],
            device_id=my, device_id_type=LOGICAL)

        def loop_wait(n, f):
            def b(i, c):
                f()
                return c
            lax.fori_loop(0, n, b, 0)

        loop_wait(N_DEV, c_wait.wait_recv)
        loop_wait(N_DEV, c_wait.wait_send)

        cv = counts_vmem[...]
        rowid = lax.broadcasted_iota(jnp.int32, (N_DEV, E), 0)
        offs = jnp.sum(jnp.where(rowid < my, cv, 0), axis=0)
        tots = jnp.sum(cv, axis=0)
        stage_vmem[...] = jnp.concatenate([offs[None, :], tots[None, :]], 0)

        rcopy.wait()
        scopy = pltpu.make_async_copy(stage_vmem, stage_smem, local_sem)
        scopy.start()
        scopy.wait()

        def dis(i, km):
            e = route_smem[i, 0]
            c = rank_ctr[e]
            rank_ctr[e] = c + 1
            r = stage_smem[0, e] + c
            keep = r < CAP
            dev = e // E_LOC
            slot = e % E_LOC

            @pl.when(keep)
            def _():
                row = slot * CAP_PAD + r
                pltpu.make_async_remote_copy(
                    x_ref.at[pl.ds(i, 1), :], xbuf.at[pl.ds(row, 1), :],
                    sems.at[X_SEND], sems.at[X_RECV],
                    device_id=dev, device_id_type=LOGICAL,
                ).start()
                pltpu.make_async_remote_copy(
                    meta_src.at[pl.ds(i, 1), :], meta_vmem.at[pl.ds(row, 1), :],
                    sems.at[M_SEND], sems.at[M_RECV],
                    device_id=dev, device_id_type=LOGICAL,
                ).start()

            return km + keep.astype(jnp.int32)

        kept_mine = lax.fori_loop(0, T, dis, 0)

        kept_slot = [jnp.minimum(stage_smem[1, my * E_LOC + s], CAP)
                     for s in range(E_LOC)]
        kept_owner = kept_slot[0] + kept_slot[1] + kept_slot[2] + kept_slot[3]

        x_wait = pltpu.make_async_remote_copy(
            x_ref.at[pl.ds(0, 1), :], xbuf.at[pl.ds(0, 1), :],
            sems.at[X_SEND], sems.at[X_RECV],
            device_id=my, device_id_type=LOGICAL)
        m_wait = pltpu.make_async_remote_copy(
            meta_src.at[pl.ds(0, 1), :], meta_vmem.at[pl.ds(0, 1), :],
            sems.at[M_SEND], sems.at[M_RECV],
            device_id=my, device_id_type=LOGICAL)
        y_wait = pltpu.make_async_remote_copy(
            ybuf.at[pl.ds(0, 1), :], out_ref.at[pl.ds(0, 1), :],
            sems.at[Y_SEND], sems.at[Y_RECV],
            device_id=my, device_id_type=LOGICAL)

        loop_wait(kept_owner, x_wait.wait_recv)
        loop_wait(kept_owner, m_wait.wait_recv)

        for s in range(E_LOC):
            a = xbuf[s * CAP_PAD:(s + 1) * CAP_PAD, :]
            ybuf[s * CAP_PAD:(s + 1) * CAP_PAD, :] = jnp.dot(
                a, w_ref[s], preferred_element_type=jnp.float32)

        mcopy = pltpu.make_async_copy(meta_vmem, meta_smem, local_sem)
        mcopy.start()
        mcopy.wait()

        for s in range(E_LOC):
            base = s * CAP_PAD

            def comb(rr, c, base=base):
                m = meta_smem[base + rr, 0]
                pltpu.make_async_remote_copy(
                    ybuf.at[pl.ds(base + rr, 1), :],
                    out_ref.at[pl.ds(m % T, 1), :],
                    sems.at[Y_SEND], sems.at[Y_RECV],
                    device_id=m // T, device_id_type=LOGICAL,
                ).start()
                return c
            lax.fori_loop(0, kept_slot[s], comb, 0)

        loop_wait(kept_mine, x_wait.wait_send)
        loop_wait(kept_mine, m_wait.wait_send)
        loop_wait(kept_owner, y_wait.wait_send)
        loop_wait(kept_mine, y_wait.wait_recv)

        def esig(d, c):
            @pl.when(d != my)
            def _():
                pl.semaphore_signal(exit_sem, 1, device_id=d,
                                    device_id_type=LOGICAL)
            return c
        lax.fori_loop(0, N_DEV, esig, 0)
        pl.semaphore_wait(exit_sem, N_DEV - 1)

    return pl.pallas_call(
        body,
        out_shape=jax.ShapeDtypeStruct((T, H), jnp.float32),
        in_specs=[pl.BlockSpec(memory_space=pltpu.VMEM)] * 3,
        out_specs=pl.BlockSpec(memory_space=pltpu.VMEM),
        scratch_shapes=[
            pltpu.VMEM((1, E), jnp.int32),
            pltpu.VMEM((N_DEV, E), jnp.int32),
            pltpu.VMEM((2, E), jnp.int32),
            pltpu.VMEM((T, 1), jnp.int32),
            pltpu.VMEM((E_LOC * CAP_PAD, D), jnp.float32),
            pltpu.VMEM((E_LOC * CAP_PAD, H), jnp.float32),
            pltpu.VMEM((E_LOC * CAP_PAD, 1), jnp.int32),
            pltpu.SMEM((T, 1), jnp.int32),
            pltpu.SMEM((2, E), jnp.int32),
            pltpu.SMEM((E_LOC * CAP_PAD, 1), jnp.int32),
            pltpu.SMEM((E,), jnp.int32),
            pltpu.SemaphoreType.DMA((8,)),
            pltpu.SemaphoreType.DMA,
            pltpu.SemaphoreType.REGULAR,
        ],
        compiler_params=pltpu.CompilerParams(collective_id=0),
    )(x, route_idx, expert_W)
